# baseline (speedup 1.0000x reference)
import jax
import jax.numpy as jnp
from jax.experimental import pallas as pl


BLK = 512


def _bcast_kernel(w_ref, out_ref):
    w = w_ref[...]
    out_ref[...] = jnp.broadcast_to(w[:, None, :], out_ref.shape)


def _part(W, b_sz):
    rows, emb = W.shape
    grid = (rows // BLK,)
    return pl.pallas_call(
        _bcast_kernel,
        grid=grid,
        in_specs=[pl.BlockSpec((BLK, emb), lambda i: (i, 0))],
        out_specs=pl.BlockSpec((BLK, b_sz, emb), lambda i: (i, 0, 0)),
        out_shape=jax.ShapeDtypeStruct((rows, b_sz, emb), W.dtype),
    )(W)


def kernel(inputs, W):
    seq_len, b_sz = inputs.shape
    half = seq_len // 2
    a = _part(W[:half], b_sz)
    b = _part(W[half:seq_len], b_sz)
    return jnp.concatenate([a, b], axis=0)


# speedup vs baseline: 1.5360x; 1.5360x over previous
import functools

import jax
import jax.numpy as jnp
from jax import lax
from jax.experimental import pallas as pl
from jax.experimental.pallas import tpu as pltpu
from jax.experimental.pallas import tpu_sc as plsc

NC = 2
NS = 16
NW = NC * NS
CHUNK = 8
NBUF = 3


def _make_sc_kernel(seq_len, b_sz, emb, dtype):
    rows_per_w = seq_len // NW
    n_chunks = rows_per_w // CHUNK
    mesh = plsc.VectorSubcoreMesh(core_axis_name="c", subcore_axis_name="s")

    @functools.partial(
        pl.kernel,
        out_type=jax.ShapeDtypeStruct((seq_len, b_sz, emb), dtype),
        mesh=mesh,
        scratch_types=[
            pltpu.VMEM((NBUF, CHUNK, b_sz, emb), dtype),
            pltpu.SemaphoreType.DMA,
            pltpu.SemaphoreType.DMA,
            pltpu.SemaphoreType.DMA,
        ],
    )
    def sc_kernel(w_hbm, out_hbm, buf, rsem, wsem, w0sem):
        wid = lax.axis_index("s") * NC + lax.axis_index("c")
        base = wid * rows_per_w

        def read(c):
            return [
                pltpu.async_copy(
                    w_hbm.at[pl.ds(base + c * CHUNK, CHUNK)],
                    buf.at[c % NBUF, :, b],
                    rsem,
                )
                for b in range(b_sz)
            ]

        def write(c, sem):
            return pltpu.async_copy(
                buf.at[c % NBUF],
                out_hbm.at[pl.ds(base + c * CHUNK, CHUNK)],
                sem,
            )

        n_reused = max(0, n_chunks - NBUF)
        rds = {}
        wrs = {}
        for c in range(min(NBUF, n_chunks)):
            rds[c] = read(c)
        for c in range(n_chunks):
            if c >= NBUF:
                wrs[c - NBUF].wait()
                rds[c] = read(c)
            for d in rds[c]:
                d.wait()
            wrs[c] = write(c, w0sem if c < n_reused else wsem)
        for c in range(n_reused, n_chunks):
            wrs[c].wait()

    return sc_kernel


def kernel(inputs, W):
    seq_len, b_sz = inputs.shape
    emb = W.shape[1]
    return _make_sc_kernel(seq_len, b_sz, emb, W.dtype)(W[:seq_len])


# speedup vs baseline: 2.4329x; 1.5839x over previous
import functools

import jax
import jax.numpy as jnp
from jax import lax
from jax.experimental import pallas as pl
from jax.experimental.pallas import tpu as pltpu
from jax.experimental.pallas import tpu_sc as plsc

NC = 2
NS = 16
NW = NC * NS
CHUNK = 16


NBUF = 6


def _make_sc_kernel(seq_len, b_sz, emb, dtype):
    rows_per_w = seq_len // NW
    n_chunks = rows_per_w // CHUNK
    mesh = plsc.VectorSubcoreMesh(core_axis_name="c", subcore_axis_name="s")

    @functools.partial(
        pl.kernel,
        out_type=jax.ShapeDtypeStruct((seq_len, b_sz, emb), dtype),
        mesh=mesh,
        scratch_types=[
            pltpu.VMEM((NBUF, CHUNK, emb), dtype),
            pltpu.SemaphoreType.DMA,
            pltpu.SemaphoreType.DMA,
            pltpu.SemaphoreType.DMA,
        ],
    )
    def sc_kernel(w_hbm, out_hbm, buf, rsem, wsem, w0sem):
        wid = lax.axis_index("s") * NC + lax.axis_index("c")
        base = wid * rows_per_w

        def read(c):
            return pltpu.async_copy(
                w_hbm.at[pl.ds(base + c * CHUNK, CHUNK)], buf.at[c % NBUF], rsem
            )

        def writes(c, sem):
            return [
                pltpu.async_copy(
                    buf.at[c % NBUF],
                    out_hbm.at[pl.ds(base + c * CHUNK, CHUNK), b],
                    sem,
                )
                for b in range(b_sz)
            ]

        n_reused = max(0, n_chunks - NBUF)
        rds = {}
        wrs = {}
        for c in range(min(NBUF, n_chunks)):
            rds[c] = read(c)
        for c in range(n_chunks):
            if c >= NBUF:
                for d in wrs[c - NBUF]:
                    d.wait()
                rds[c] = read(c)
            rds[c].wait()
            wrs[c] = writes(c, w0sem if c < n_reused else wsem)
        for c in range(n_reused, n_chunks):
            for d in wrs[c]:
                d.wait()

    return sc_kernel


def kernel(inputs, W):
    seq_len, b_sz = inputs.shape
    emb = W.shape[1]
    return _make_sc_kernel(seq_len, b_sz, emb, W.dtype)(W[:seq_len])


# speedup vs baseline: 2.5552x; 1.0503x over previous
import functools

import jax
import jax.numpy as jnp
from jax import lax
from jax.experimental import pallas as pl
from jax.experimental.pallas import tpu as pltpu
from jax.experimental.pallas import tpu_sc as plsc

NC = 2
NS = 16
NW = NC * NS
CHUNK = 32
NBUF = 2


def _make_sc_kernel(seq_len, b_sz, emb, dtype):
    rows_per_w = seq_len // NW
    n_chunks = rows_per_w // CHUNK
    mesh = plsc.VectorSubcoreMesh(core_axis_name="c", subcore_axis_name="s")

    @functools.partial(
        pl.kernel,
        out_type=jax.ShapeDtypeStruct((seq_len, b_sz, emb), dtype),
        mesh=mesh,
        scratch_types=[
            pltpu.VMEM((NBUF, CHUNK, emb), dtype),
            pltpu.SemaphoreType.DMA,
            pltpu.SemaphoreType.DMA,
        ],
    )
    def sc_kernel(w_hbm, out_hbm, buf, rsem, wsem):
        wid = lax.axis_index("s") * NC + lax.axis_index("c")
        base = wid * rows_per_w

        def read(c):
            return pltpu.async_copy(
                w_hbm.at[pl.ds(base + c * CHUNK, CHUNK)], buf.at[c % NBUF], rsem
            )

        def writes(c):
            return [
                pltpu.async_copy(
                    buf.at[c % NBUF],
                    out_hbm.at[pl.ds(base + c * CHUNK, CHUNK), b],
                    wsem,
                )
                for b in range(b_sz)
            ]

        rds = {}
        wrs = {}
        rds[0] = read(0)
        if n_chunks > 1:
            rds[1] = read(1)
        for c in range(n_chunks):
            rds[c].wait()
            wrs[c] = writes(c)
            if c + NBUF < n_chunks:
                for d in wrs[c]:
                    d.wait()
                rds[c + NBUF] = read(c + NBUF)
        for c in range(max(0, n_chunks - NBUF), n_chunks):
            for d in wrs[c]:
                d.wait()

    return sc_kernel


def kernel(inputs, W):
    seq_len, b_sz = inputs.shape
    emb = W.shape[1]
    return _make_sc_kernel(seq_len, b_sz, emb, W.dtype)(W[:seq_len])


# speedup vs baseline: 2.5662x; 1.0043x over previous
import functools

import jax
import jax.numpy as jnp
from jax import lax
from jax.experimental import pallas as pl
from jax.experimental.pallas import tpu as pltpu
from jax.experimental.pallas import tpu_sc as plsc

NC = 2
NS = 16
NW = NC * NS
CHUNK = 32
NBUF = 2


def _make_sc_kernel(seq_len, b_sz, emb, dtype):
    rows_per_w = seq_len // NW
    chunk_rows = [64, 56, 8]
    chunk_off = [0, 64, 120]
    buf_off = [0, 64, 0]
    assert sum(chunk_rows) == rows_per_w
    mesh = plsc.VectorSubcoreMesh(core_axis_name="c", subcore_axis_name="s")

    @functools.partial(
        pl.kernel,
        out_type=jax.ShapeDtypeStruct((seq_len, b_sz, emb), dtype),
        mesh=mesh,
        scratch_types=[
            pltpu.VMEM((120, emb), dtype),
            pltpu.SemaphoreType.DMA,
            pltpu.SemaphoreType.DMA,
            pltpu.SemaphoreType.DMA,
        ],
    )
    def sc_kernel(w_hbm, out_hbm, buf, rsem, wsem, w0sem):
        wid = lax.axis_index("s") * NC + lax.axis_index("c")
        base = wid * rows_per_w

        def read(c):
            return pltpu.async_copy(
                w_hbm.at[pl.ds(base + chunk_off[c], chunk_rows[c])],
                buf.at[pl.ds(buf_off[c], chunk_rows[c])],
                rsem,
            )

        def writes(c, sem):
            return [
                pltpu.async_copy(
                    buf.at[pl.ds(buf_off[c], chunk_rows[c])],
                    out_hbm.at[pl.ds(base + chunk_off[c], chunk_rows[c]), b],
                    sem,
                )
                for b in range(b_sz)
            ]

        rds = {0: read(0), 1: read(1)}
        rds[0].wait()
        w0 = writes(0, w0sem)
        rds[1].wait()
        w1 = writes(1, wsem)
        for d in w0:
            d.wait()
        rds[2] = read(2)
        rds[2].wait()
        w2 = writes(2, wsem)
        for d in w1 + w2:
            d.wait()

    return sc_kernel


def kernel(inputs, W):
    seq_len, b_sz = inputs.shape
    emb = W.shape[1]
    return _make_sc_kernel(seq_len, b_sz, emb, W.dtype)(W[:seq_len])
